# Initial kernel scaffold; baseline (speedup 1.0000x reference)
import jax
import jax.numpy as jnp
from jax.experimental import pallas as pl


def kernel(oov, read_depth, covariates, extra_features, emb_table, W1, b1, W2, b2, iv):
    raise NotImplementedError("write your pallas kernel here")


# speedup vs baseline: 2.2801x; 2.2801x over previous
import functools

import jax
import jax.numpy as jnp
from jax import lax
from jax.experimental import pallas as pl
from jax.experimental.pallas import tpu as pltpu
from jax.experimental.pallas import tpu_sc as plsc

B = 4096
L = 200
DIM = 32
OOV = 512
NCOV = 8
NX = 24
HID = 512
NTOP = 64

NC, NS = 2, 16
NW = NC * NS
RPW = B // NW
IDX0, IDX1 = 128, L - 128

SMALL = 1 + NCOV + DIM + NX
SPAD = 128
TILE = 512


def _dan_body(iv_hbm, table_hbm, out_hbm, idx_v, rows_v, out_v, sem0, sem1):
    wid = lax.axis_index("s") * NC + lax.axis_index("c")
    base = wid * (RPW * L)
    pltpu.sync_copy(iv_hbm.at[pl.ds(base, RPW * L)], idx_v.at[pl.ds(0, RPW * L)])

    def gather_descs(r, slot_ref, sem):
        off = r * L
        d0 = pltpu.make_async_copy(
            table_hbm.at[idx_v.at[pl.ds(off, IDX0)]],
            slot_ref.at[pl.ds(0, IDX0)], sem)
        d1 = pltpu.make_async_copy(
            table_hbm.at[idx_v.at[pl.ds(off + IDX0, IDX1)]],
            slot_ref.at[pl.ds(IDX0, IDX1)], sem)
        return d0, d1

    def issue(r, slot_ref, sem):
        d0, d1 = gather_descs(r, slot_ref, sem)
        d0.start()
        d1.start()

    lanes = lax.iota(jnp.int32, 16)

    def process(r, slot_ref, sem):
        d0, d1 = gather_descs(r, slot_ref, sem)
        d0.wait()
        d1.wait()
        ivoff = r * L
        accc = jnp.zeros((16,), jnp.int32)
        for k in range(12):
            v = idx_v[pl.ds(ivoff + 16 * k, 16)]
            accc = accc + plsc.all_reduce_population_count(v > 0)
        v = idx_v[pl.ds(ivoff + 192, 16)]
        accc = accc + plsc.all_reduce_population_count(
            (v > 0) & (lanes < L - 192))
        inv = 1.0 / jnp.maximum(accc.astype(jnp.float32), 1.0)

        def add_row(j, accs):
            a0, a1 = accs
            a0 = a0 + slot_ref[j, pl.ds(0, 16)]
            a1 = a1 + slot_ref[j, pl.ds(16, 16)]
            return (a0, a1)

        acc0 = jnp.zeros((16,), jnp.float32)
        acc1 = jnp.zeros((16,), jnp.float32)
        acc0, acc1 = lax.fori_loop(0, L, add_row, (acc0, acc1), unroll=8)
        out_v[r, pl.ds(0, 16)] = acc0 * inv
        out_v[r, pl.ds(16, 16)] = acc1 * inv

    issue(0, rows_v.at[0], sem0)
    issue(1, rows_v.at[1], sem1)

    def pair_body(i, carry):
        r0 = 2 * i
        process(r0, rows_v.at[0], sem0)

        @pl.when(r0 + 2 < RPW)
        def _():
            issue(r0 + 2, rows_v.at[0], sem0)

        process(r0 + 1, rows_v.at[1], sem1)

        @pl.when(r0 + 3 < RPW)
        def _():
            issue(r0 + 3, rows_v.at[1], sem1)

        return carry

    lax.fori_loop(0, RPW // 2, pair_body, 0)
    pltpu.sync_copy(out_v, out_hbm.at[pl.ds(wid * RPW, RPW)])


@functools.cache
def _get_dan_sc():
    return pl.kernel(
        _dan_body,
        out_type=jax.ShapeDtypeStruct((B, DIM), jnp.float32),
        mesh=plsc.VectorSubcoreMesh(core_axis_name="c", subcore_axis_name="s",
                                    num_cores=NC, num_subcores=NS),
        compiler_params=pltpu.CompilerParams(needs_layout_passes=False,
                                             use_tc_tiling_on_sc=False),
        scratch_types=[
            pltpu.VMEM((RPW * L + 8,), jnp.int32),
            pltpu.VMEM((2, L, DIM), jnp.float32),
            pltpu.VMEM((RPW, DIM), jnp.float32),
            pltpu.SemaphoreType.DMA,
            pltpu.SemaphoreType.DMA,
        ],
    )


def _mlp_body(oov_ref, small_ref, w1a_ref, w1b_ref, b1_ref, w2_ref, b2_ref,
              out_ref):
    small = small_ref[...]
    col = lax.broadcasted_iota(jnp.int32, small.shape, 1)
    small = jnp.where(col == 0, jnp.log1p(small), small)
    x = jnp.dot(oov_ref[...], w1a_ref[...], preferred_element_type=jnp.float32)
    x = x + jnp.dot(small, w1b_ref[...], preferred_element_type=jnp.float32)
    h = jnp.maximum(x + b1_ref[...], 0.0)
    out_ref[...] = (
        jnp.dot(h, w2_ref[...], preferred_element_type=jnp.float32)
        + b2_ref[...])


def _mlp_tc(oov, small, w1a, w1b, b1, w2, b2):
    grid = (B // TILE,)
    return pl.pallas_call(
        _mlp_body,
        grid=grid,
        in_specs=[
            pl.BlockSpec((TILE, OOV), lambda i: (i, 0)),
            pl.BlockSpec((TILE, SPAD), lambda i: (i, 0)),
            pl.BlockSpec((OOV, HID), lambda i: (0, 0)),
            pl.BlockSpec((SPAD, HID), lambda i: (0, 0)),
            pl.BlockSpec((1, HID), lambda i: (0, 0)),
            pl.BlockSpec((HID, NTOP), lambda i: (0, 0)),
            pl.BlockSpec((1, NTOP), lambda i: (0, 0)),
        ],
        out_specs=pl.BlockSpec((TILE, NTOP), lambda i: (i, 0)),
        out_shape=jax.ShapeDtypeStruct((B, NTOP), jnp.float32),
    )(oov, small, w1a, w1b, b1, w2, b2)


def kernel(oov, read_depth, covariates, extra_features, emb_table, W1, b1,
           W2, b2, iv):
    ivf = iv.reshape(-1).astype(jnp.int32)
    iv_rep = _get_dan_sc()(ivf, emb_table)
    small = jnp.concatenate(
        [read_depth, covariates, iv_rep, extra_features], axis=1)
    small = jnp.pad(small, ((0, 0), (0, SPAD - SMALL)))
    w1a = W1[:OOV]
    w1b = jnp.pad(W1[OOV:], ((0, SPAD - SMALL), (0, 0)))
    return _mlp_tc(oov, small, w1a, w1b, b1.reshape(1, -1), W2,
                   b2.reshape(1, -1))


# speedup vs baseline: 2.5801x; 1.1315x over previous
import functools

import jax
import jax.numpy as jnp
from jax import lax
from jax.experimental import pallas as pl
from jax.experimental.pallas import tpu as pltpu
from jax.experimental.pallas import tpu_sc as plsc

B = 4096
L = 200
DIM = 32
OOV = 512
NCOV = 8
NX = 24
HID = 512
NTOP = 64
VOCAB1 = 1000001

NC, NS = 2, 16
NW = NC * NS
RPW = B // NW
IDX0, IDX1 = 128, L - 128

TPW = 2048
NBLK = (VOCAB1 + TPW - 1) // TPW
NROWS = NBLK * TPW

SMALL = 1 + NCOV + DIM + NX
SPAD = 128
TILE = 512


def _repack_body(in_ref, out_ref):
    for i in range(4):
        out_ref[:, 32 * i:32 * i + 32] = jnp.transpose(
            in_ref[:, 512 * i:512 * i + 512])


def _repack_tc(tblT):
    return pl.pallas_call(
        _repack_body,
        grid=(NBLK,),
        in_specs=[pl.BlockSpec((32, TPW), lambda g: (0, g))],
        out_specs=pl.BlockSpec((512, 128), lambda g: (g, 0)),
        out_shape=jax.ShapeDtypeStruct((NBLK * 512, 128), jnp.float32),
    )(tblT)


def _dan_body(iv_hbm, table_hbm, out_hbm, idx_v, rows_v, out_v, sem0, sem1):
    wid = lax.axis_index("s") * NC + lax.axis_index("c")
    base = wid * (RPW * L)
    pltpu.sync_copy(iv_hbm.at[pl.ds(base, RPW * L)], idx_v.at[pl.ds(0, RPW * L)])

    def xform(k, carry):
        t = idx_v[pl.ds(16 * k, 16)]
        row = ((t & jnp.int32(~2047))
               + ((t & jnp.int32(511)) << 2)
               + ((t >> 9) & jnp.int32(3)))
        idx_v[pl.ds(16 * k, 16)] = row
        return carry

    lax.fori_loop(0, (RPW * L) // 16, xform, 0, unroll=8)

    def gather_descs(r, slot_ref, sem):
        off = r * L
        d0 = pltpu.make_async_copy(
            table_hbm.at[idx_v.at[pl.ds(off, IDX0)]],
            slot_ref.at[pl.ds(0, IDX0)], sem)
        d1 = pltpu.make_async_copy(
            table_hbm.at[idx_v.at[pl.ds(off + IDX0, IDX1)]],
            slot_ref.at[pl.ds(IDX0, IDX1)], sem)
        return d0, d1

    def issue(r, slot_ref, sem):
        d0, d1 = gather_descs(r, slot_ref, sem)
        d0.start()
        d1.start()

    lanes = lax.iota(jnp.int32, 16)

    def process(r, slot_ref, sem):
        d0, d1 = gather_descs(r, slot_ref, sem)
        d0.wait()
        d1.wait()
        ivoff = r * L
        accc = jnp.zeros((16,), jnp.int32)
        for k in range(12):
            v = idx_v[pl.ds(ivoff + 16 * k, 16)]
            accc = accc + plsc.all_reduce_population_count(v > 0)
        v = idx_v[pl.ds(ivoff + 192, 16)]
        accc = accc + plsc.all_reduce_population_count(
            (v > 0) & (lanes < L - 192))
        inv = 1.0 / jnp.maximum(accc.astype(jnp.float32), 1.0)

        def add_row(j, accs):
            a0, a1 = accs
            a0 = a0 + slot_ref[j, pl.ds(0, 16)]
            a1 = a1 + slot_ref[j, pl.ds(16, 16)]
            return (a0, a1)

        acc0 = jnp.zeros((16,), jnp.float32)
        acc1 = jnp.zeros((16,), jnp.float32)
        acc0, acc1 = lax.fori_loop(0, L, add_row, (acc0, acc1), unroll=8)
        out_v[r, pl.ds(0, 16)] = acc0 * inv
        out_v[r, pl.ds(16, 16)] = acc1 * inv

    issue(0, rows_v.at[0], sem0)
    issue(1, rows_v.at[1], sem1)

    def pair_body(i, carry):
        r0 = 2 * i
        process(r0, rows_v.at[0], sem0)

        @pl.when(r0 + 2 < RPW)
        def _():
            issue(r0 + 2, rows_v.at[0], sem0)

        process(r0 + 1, rows_v.at[1], sem1)

        @pl.when(r0 + 3 < RPW)
        def _():
            issue(r0 + 3, rows_v.at[1], sem1)

        return carry

    lax.fori_loop(0, RPW // 2, pair_body, 0)
    pltpu.sync_copy(out_v, out_hbm.at[pl.ds(wid * RPW, RPW)])


@functools.cache
def _get_dan_sc():
    return pl.kernel(
        _dan_body,
        out_type=jax.ShapeDtypeStruct((B, DIM), jnp.float32),
        mesh=plsc.VectorSubcoreMesh(core_axis_name="c", subcore_axis_name="s",
                                    num_cores=NC, num_subcores=NS),
        compiler_params=pltpu.CompilerParams(needs_layout_passes=False,
                                             use_tc_tiling_on_sc=False),
        scratch_types=[
            pltpu.VMEM((RPW * L + 8,), jnp.int32),
            pltpu.VMEM((2, L, DIM), jnp.float32),
            pltpu.VMEM((RPW, DIM), jnp.float32),
            pltpu.SemaphoreType.DMA,
            pltpu.SemaphoreType.DMA,
        ],
    )


def _mlp_body(oov_ref, small_ref, w1a_ref, w1b_ref, b1_ref, w2_ref, b2_ref,
              out_ref):
    small = small_ref[...]
    col = lax.broadcasted_iota(jnp.int32, small.shape, 1)
    small = jnp.where(col == 0, jnp.log1p(small), small)
    x = jnp.dot(oov_ref[...], w1a_ref[...], preferred_element_type=jnp.float32)
    x = x + jnp.dot(small, w1b_ref[...], preferred_element_type=jnp.float32)
    h = jnp.maximum(x + b1_ref[...], 0.0)
    out_ref[...] = (
        jnp.dot(h, w2_ref[...], preferred_element_type=jnp.float32)
        + b2_ref[...])


def _mlp_tc(oov, small, w1a, w1b, b1, w2, b2):
    grid = (B // TILE,)
    return pl.pallas_call(
        _mlp_body,
        grid=grid,
        in_specs=[
            pl.BlockSpec((TILE, OOV), lambda i: (i, 0)),
            pl.BlockSpec((TILE, SPAD), lambda i: (i, 0)),
            pl.BlockSpec((OOV, HID), lambda i: (0, 0)),
            pl.BlockSpec((SPAD, HID), lambda i: (0, 0)),
            pl.BlockSpec((1, HID), lambda i: (0, 0)),
            pl.BlockSpec((HID, NTOP), lambda i: (0, 0)),
            pl.BlockSpec((1, NTOP), lambda i: (0, 0)),
        ],
        out_specs=pl.BlockSpec((TILE, NTOP), lambda i: (i, 0)),
        out_shape=jax.ShapeDtypeStruct((B, NTOP), jnp.float32),
    )(oov, small, w1a, w1b, b1, w2, b2)


def kernel(oov, read_depth, covariates, extra_features, emb_table, W1, b1,
           W2, b2, iv):
    packed = _repack_tc(emb_table.T)
    packed_rows = packed.reshape(-1).reshape(NROWS, DIM)
    ivf = iv.reshape(-1).astype(jnp.int32)
    iv_rep = _get_dan_sc()(ivf, packed_rows)
    small = jnp.concatenate(
        [read_depth, covariates, iv_rep, extra_features], axis=1)
    small = jnp.pad(small, ((0, 0), (0, SPAD - SMALL)))
    w1a = W1[:OOV]
    w1b = jnp.pad(W1[OOV:], ((0, SPAD - SMALL), (0, 0)))
    return _mlp_tc(oov, small, w1a, w1b, b1.reshape(1, -1), W2,
                   b2.reshape(1, -1))


# speedup vs baseline: 5.1675x; 2.0029x over previous
import functools

import jax
import jax.numpy as jnp
from jax import lax
from jax.experimental import pallas as pl
from jax.experimental.pallas import tpu as pltpu
from jax.experimental.pallas import tpu_sc as plsc

B = 4096
L = 200
DIM = 32
OOV = 512
NCOV = 8
NX = 24
HID = 512
NTOP = 64
VOCAB1 = 1000001

NC, NS = 2, 16
NW = NC * NS
RPW = B // NW
IDX0, IDX1 = 128, L - 128

TPW = 8192
NBLK = (VOCAB1 + TPW - 1) // TPW
NROWS = NBLK * TPW
SLICE = TPW // 4
NCBLK = (VOCAB1 + SLICE - 1) // SLICE

SMALL = 1 + NCOV + DIM + NX
SPAD = 128
TILE = 512


def _repack_body(b0, b1, b2, b3, out_ref):
    g = pl.program_id(0)
    x = jnp.concatenate([b0[...], b1[...], b2[...], b3[...]], axis=0)
    col = lax.broadcasted_iota(jnp.int32, (4 * DIM, SLICE), 1)
    srow = lax.broadcasted_iota(jnp.int32, (4 * DIM, SLICE), 0) // DIM
    tok = g * TPW + srow * SLICE + col
    x = jnp.where(tok < VOCAB1, x, 0.0)
    row = lax.broadcasted_iota(jnp.int32, (4 * DIM, 128), 0)
    lcol = lax.broadcasted_iota(jnp.int32, (4 * DIM, 128), 1)
    eye = (row == lcol).astype(jnp.float32)
    out_ref[...] = lax.dot_general(
        x, eye, (((0,), (0,)), ((), ())),
        preferred_element_type=jnp.float32)


def _repack_tc(tblT):
    return pl.pallas_call(
        _repack_body,
        grid=(NBLK,),
        in_specs=[
            pl.BlockSpec((32, SLICE),
                         lambda g, _i=i: (0, jnp.minimum(4 * g + _i, NCBLK - 1)))
            for i in range(4)
        ],
        out_specs=pl.BlockSpec((SLICE, 128), lambda g: (g, 0)),
        out_shape=jax.ShapeDtypeStruct((NBLK * SLICE, 128), jnp.float32),
        compiler_params=pltpu.CompilerParams(
            fuse_transposed_lhs_in_matmul=True),
    )(tblT, tblT, tblT, tblT)


def _dan_body(iv_hbm, table_hbm, out_hbm, idx_v, rows_v, out_v, sem0, sem1):
    wid = lax.axis_index("s") * NC + lax.axis_index("c")
    base = wid * (RPW * L)
    pltpu.sync_copy(iv_hbm.at[pl.ds(base, RPW * L)], idx_v.at[pl.ds(0, RPW * L)])

    slice_shift = SLICE.bit_length() - 1

    def xform(k, carry):
        t = idx_v[pl.ds(16 * k, 16)]
        row = ((t & jnp.int32(~(TPW - 1)))
               + ((t & jnp.int32(SLICE - 1)) << 2)
               + ((t >> slice_shift) & jnp.int32(3)))
        idx_v[pl.ds(16 * k, 16)] = row
        return carry

    lax.fori_loop(0, (RPW * L) // 16, xform, 0, unroll=8)

    def gather_descs(r, slot_ref, sem):
        off = r * L
        d0 = pltpu.make_async_copy(
            table_hbm.at[idx_v.at[pl.ds(off, IDX0)]],
            slot_ref.at[pl.ds(0, IDX0)], sem)
        d1 = pltpu.make_async_copy(
            table_hbm.at[idx_v.at[pl.ds(off + IDX0, IDX1)]],
            slot_ref.at[pl.ds(IDX0, IDX1)], sem)
        return d0, d1

    def issue(r, slot_ref, sem):
        d0, d1 = gather_descs(r, slot_ref, sem)
        d0.start()
        d1.start()

    lanes = lax.iota(jnp.int32, 16)

    def process(r, slot_ref, sem):
        d0, d1 = gather_descs(r, slot_ref, sem)
        d0.wait()
        d1.wait()
        ivoff = r * L
        accc = jnp.zeros((16,), jnp.int32)
        for k in range(12):
            v = idx_v[pl.ds(ivoff + 16 * k, 16)]
            accc = accc + plsc.all_reduce_population_count(v > 0)
        v = idx_v[pl.ds(ivoff + 192, 16)]
        accc = accc + plsc.all_reduce_population_count(
            (v > 0) & (lanes < L - 192))
        inv = 1.0 / jnp.maximum(accc.astype(jnp.float32), 1.0)

        def add_row(j, accs):
            a0, a1 = accs
            a0 = a0 + slot_ref[j, pl.ds(0, 16)]
            a1 = a1 + slot_ref[j, pl.ds(16, 16)]
            return (a0, a1)

        acc0 = jnp.zeros((16,), jnp.float32)
        acc1 = jnp.zeros((16,), jnp.float32)
        acc0, acc1 = lax.fori_loop(0, L, add_row, (acc0, acc1), unroll=8)
        out_v[r, pl.ds(0, 16)] = acc0 * inv
        out_v[r, pl.ds(16, 16)] = acc1 * inv

    issue(0, rows_v.at[0], sem0)
    issue(1, rows_v.at[1], sem1)

    def pair_body(i, carry):
        r0 = 2 * i
        process(r0, rows_v.at[0], sem0)

        @pl.when(r0 + 2 < RPW)
        def _():
            issue(r0 + 2, rows_v.at[0], sem0)

        process(r0 + 1, rows_v.at[1], sem1)

        @pl.when(r0 + 3 < RPW)
        def _():
            issue(r0 + 3, rows_v.at[1], sem1)

        return carry

    lax.fori_loop(0, RPW // 2, pair_body, 0)
    pltpu.sync_copy(out_v, out_hbm.at[pl.ds(wid * RPW, RPW)])


@functools.cache
def _get_dan_sc():
    return pl.kernel(
        _dan_body,
        out_type=jax.ShapeDtypeStruct((B, DIM), jnp.float32),
        mesh=plsc.VectorSubcoreMesh(core_axis_name="c", subcore_axis_name="s",
                                    num_cores=NC, num_subcores=NS),
        compiler_params=pltpu.CompilerParams(needs_layout_passes=False,
                                             use_tc_tiling_on_sc=False),
        scratch_types=[
            pltpu.VMEM((RPW * L + 8,), jnp.int32),
            pltpu.VMEM((2, L, DIM), jnp.float32),
            pltpu.VMEM((RPW, DIM), jnp.float32),
            pltpu.SemaphoreType.DMA,
            pltpu.SemaphoreType.DMA,
        ],
    )


def _mlp_body(oov_ref, small_ref, w1a_ref, w1b_ref, b1_ref, w2_ref, b2_ref,
              out_ref):
    small = small_ref[...]
    col = lax.broadcasted_iota(jnp.int32, small.shape, 1)
    small = jnp.where(col == 0, jnp.log1p(small), small)
    x = jnp.dot(oov_ref[...], w1a_ref[...], preferred_element_type=jnp.float32)
    x = x + jnp.dot(small, w1b_ref[...], preferred_element_type=jnp.float32)
    h = jnp.maximum(x + b1_ref[...], 0.0)
    out_ref[...] = (
        jnp.dot(h, w2_ref[...], preferred_element_type=jnp.float32)
        + b2_ref[...])


def _mlp_tc(oov, small, w1a, w1b, b1, w2, b2):
    grid = (B // TILE,)
    return pl.pallas_call(
        _mlp_body,
        grid=grid,
        in_specs=[
            pl.BlockSpec((TILE, OOV), lambda i: (i, 0)),
            pl.BlockSpec((TILE, SPAD), lambda i: (i, 0)),
            pl.BlockSpec((OOV, HID), lambda i: (0, 0)),
            pl.BlockSpec((SPAD, HID), lambda i: (0, 0)),
            pl.BlockSpec((1, HID), lambda i: (0, 0)),
            pl.BlockSpec((HID, NTOP), lambda i: (0, 0)),
            pl.BlockSpec((1, NTOP), lambda i: (0, 0)),
        ],
        out_specs=pl.BlockSpec((TILE, NTOP), lambda i: (i, 0)),
        out_shape=jax.ShapeDtypeStruct((B, NTOP), jnp.float32),
    )(oov, small, w1a, w1b, b1, w2, b2)


def kernel(oov, read_depth, covariates, extra_features, emb_table, W1, b1,
           W2, b2, iv):
    packed = _repack_tc(emb_table.T)
    packed_rows = packed.reshape(-1).reshape(NROWS, DIM)
    ivf = iv.reshape(-1).astype(jnp.int32)
    iv_rep = _get_dan_sc()(ivf, packed_rows)
    small = jnp.concatenate(
        [read_depth, covariates, iv_rep, extra_features], axis=1)
    small = jnp.pad(small, ((0, 0), (0, SPAD - SMALL)))
    w1a = W1[:OOV]
    w1b = jnp.pad(W1[OOV:], ((0, SPAD - SMALL), (0, 0)))
    return _mlp_tc(oov, small, w1a, w1b, b1.reshape(1, -1), W2,
                   b2.reshape(1, -1))


# speedup vs baseline: 5.6013x; 1.0839x over previous
import functools

import jax
import jax.numpy as jnp
from jax import lax
from jax.experimental import pallas as pl
from jax.experimental.pallas import tpu as pltpu
from jax.experimental.pallas import tpu_sc as plsc

B = 4096
L = 200
DIM = 32
OOV = 512
NCOV = 8
NX = 24
HID = 512
NTOP = 64
VOCAB1 = 1000001

NC, NS = 2, 16
NW = NC * NS
RPW = B // NW
IDX0, IDX1 = 128, L - 128

TPW = 8192
NBLK = (VOCAB1 + TPW - 1) // TPW
NROWS = NBLK * TPW
CH = TPW // 8
NCBLK = (VOCAB1 + CH - 1) // CH

SMALL = 1 + NCOV + DIM + NX
SPAD = 128
TILE = 512


def _repack_body(b0, b1, b2, b3, b4, b5, b6, b7, out_ref):
    g = pl.program_id(0)
    blocks = (b0, b1, b2, b3, b4, b5, b6, b7)
    xh = jnp.concatenate([b[0:16, :] for b in blocks], axis=0)
    xl = jnp.concatenate([b[16:32, :] for b in blocks], axis=0)
    col = lax.broadcasted_iota(jnp.int32, (128, CH), 1)
    p_in = lax.broadcasted_iota(jnp.int32, (128, CH), 0) // 16
    tok = g * TPW + p_in * CH + col
    valid = tok < VOCAB1
    xh = jnp.where(valid, xh, 0.0)
    xl = jnp.where(valid, xl, 0.0)

    row = lax.broadcasted_iota(jnp.int32, (128, 128), 0)
    lcol = lax.broadcasted_iota(jnp.int32, (128, 128), 1)
    eye = (row == lcol).astype(jnp.float32)
    a = lax.dot_general(xh, eye, (((0,), (0,)), ((), ())),
                        preferred_element_type=jnp.float32)
    bm = lax.dot_general(xl, eye, (((0,), (0,)), ((), ())),
                         preferred_element_type=jnp.float32)
    ua = lax.bitcast_convert_type(a, jnp.int32)
    ub = lax.bitcast_convert_type(bm, jnp.int32)
    hi = (ua + jnp.int32(0x8000)) & jnp.int32(-65536)
    lo = lax.shift_right_logical(ub + jnp.int32(0x8000), 16)
    out_ref[...] = lax.bitcast_convert_type(hi | lo, jnp.float32)


def _repack_tc(tblT):
    return pl.pallas_call(
        _repack_body,
        grid=(NBLK,),
        in_specs=[
            pl.BlockSpec((32, CH),
                         lambda g, _i=i: (0, jnp.minimum(8 * g + _i, NCBLK - 1)))
            for i in range(8)
        ],
        out_specs=pl.BlockSpec((CH, 128), lambda g: (g, 0)),
        out_shape=jax.ShapeDtypeStruct((NBLK * CH, 128), jnp.float32),
        compiler_params=pltpu.CompilerParams(
            fuse_transposed_lhs_in_matmul=True),
    )(*([tblT] * 8))


def _dan_body(iv_hbm, table_hbm, out_hbm, idx_v, rows_v, out_v, sem0, sem1):
    wid = lax.axis_index("s") * NC + lax.axis_index("c")
    base = wid * (RPW * L)
    pltpu.sync_copy(iv_hbm.at[pl.ds(base, RPW * L)], idx_v.at[pl.ds(0, RPW * L)])

    ch_shift = CH.bit_length() - 1

    def xform(k, carry):
        t = idx_v[pl.ds(16 * k, 16)]
        row = ((t & jnp.int32(~(TPW - 1)))
               + ((t & jnp.int32(CH - 1)) << 3)
               + ((t >> ch_shift) & jnp.int32(7)))
        idx_v[pl.ds(16 * k, 16)] = row
        return carry

    lax.fori_loop(0, (RPW * L) // 16, xform, 0, unroll=8)

    def gather_descs(r, slot_ref, sem):
        off = r * L
        d0 = pltpu.make_async_copy(
            table_hbm.at[idx_v.at[pl.ds(off, IDX0)]],
            slot_ref.at[pl.ds(0, IDX0)], sem)
        d1 = pltpu.make_async_copy(
            table_hbm.at[idx_v.at[pl.ds(off + IDX0, IDX1)]],
            slot_ref.at[pl.ds(IDX0, IDX1)], sem)
        return d0, d1

    def issue(r, slot_ref, sem):
        d0, d1 = gather_descs(r, slot_ref, sem)
        d0.start()
        d1.start()

    lanes = lax.iota(jnp.int32, 16)

    def process(r, slot_ref, sem):
        d0, d1 = gather_descs(r, slot_ref, sem)
        d0.wait()
        d1.wait()
        ivoff = r * L
        accc = jnp.zeros((16,), jnp.int32)
        for k in range(12):
            v = idx_v[pl.ds(ivoff + 16 * k, 16)]
            accc = accc + plsc.all_reduce_population_count(v > 0)
        v = idx_v[pl.ds(ivoff + 192, 16)]
        accc = accc + plsc.all_reduce_population_count(
            (v > 0) & (lanes < L - 192))
        inv = 1.0 / jnp.maximum(accc.astype(jnp.float32), 1.0)

        def add_row(j, accs):
            a0, a1 = accs
            w = plsc.bitcast(slot_ref[j, pl.ds(0, 16)], jnp.int32)
            a0 = a0 + plsc.bitcast(w & jnp.int32(-65536), jnp.float32)
            a1 = a1 + plsc.bitcast(w << 16, jnp.float32)
            return (a0, a1)

        acc0 = jnp.zeros((16,), jnp.float32)
        acc1 = jnp.zeros((16,), jnp.float32)
        acc0, acc1 = lax.fori_loop(0, L, add_row, (acc0, acc1), unroll=8)
        out_v[r, pl.ds(0, 16)] = acc0 * inv
        out_v[r, pl.ds(16, 16)] = acc1 * inv

    issue(0, rows_v.at[0], sem0)
    issue(1, rows_v.at[1], sem1)

    def pair_body(i, carry):
        r0 = 2 * i
        process(r0, rows_v.at[0], sem0)

        @pl.when(r0 + 2 < RPW)
        def _():
            issue(r0 + 2, rows_v.at[0], sem0)

        process(r0 + 1, rows_v.at[1], sem1)

        @pl.when(r0 + 3 < RPW)
        def _():
            issue(r0 + 3, rows_v.at[1], sem1)

        return carry

    lax.fori_loop(0, RPW // 2, pair_body, 0)
    pltpu.sync_copy(out_v, out_hbm.at[pl.ds(wid * RPW, RPW)])


@functools.cache
def _get_dan_sc():
    return pl.kernel(
        _dan_body,
        out_type=jax.ShapeDtypeStruct((B, DIM), jnp.float32),
        mesh=plsc.VectorSubcoreMesh(core_axis_name="c", subcore_axis_name="s",
                                    num_cores=NC, num_subcores=NS),
        compiler_params=pltpu.CompilerParams(needs_layout_passes=False,
                                             use_tc_tiling_on_sc=False),
        scratch_types=[
            pltpu.VMEM((RPW * L + 8,), jnp.int32),
            pltpu.VMEM((2, L, DIM // 2), jnp.float32),
            pltpu.VMEM((RPW, DIM), jnp.float32),
            pltpu.SemaphoreType.DMA,
            pltpu.SemaphoreType.DMA,
        ],
    )


def _mlp_body(oov_ref, small_ref, w1a_ref, w1b_ref, b1_ref, w2_ref, b2_ref,
              out_ref):
    small = small_ref[...]
    col = lax.broadcasted_iota(jnp.int32, small.shape, 1)
    small = jnp.where(col == 0, jnp.log1p(small), small)
    x = jnp.dot(oov_ref[...], w1a_ref[...], preferred_element_type=jnp.float32)
    x = x + jnp.dot(small, w1b_ref[...], preferred_element_type=jnp.float32)
    h = jnp.maximum(x + b1_ref[...], 0.0)
    out_ref[...] = (
        jnp.dot(h, w2_ref[...], preferred_element_type=jnp.float32)
        + b2_ref[...])


def _mlp_tc(oov, small, w1a, w1b, b1, w2, b2):
    grid = (B // TILE,)
    return pl.pallas_call(
        _mlp_body,
        grid=grid,
        in_specs=[
            pl.BlockSpec((TILE, OOV), lambda i: (i, 0)),
            pl.BlockSpec((TILE, SPAD), lambda i: (i, 0)),
            pl.BlockSpec((OOV, HID), lambda i: (0, 0)),
            pl.BlockSpec((SPAD, HID), lambda i: (0, 0)),
            pl.BlockSpec((1, HID), lambda i: (0, 0)),
            pl.BlockSpec((HID, NTOP), lambda i: (0, 0)),
            pl.BlockSpec((1, NTOP), lambda i: (0, 0)),
        ],
        out_specs=pl.BlockSpec((TILE, NTOP), lambda i: (i, 0)),
        out_shape=jax.ShapeDtypeStruct((B, NTOP), jnp.float32),
    )(oov, small, w1a, w1b, b1, w2, b2)


def kernel(oov, read_depth, covariates, extra_features, emb_table, W1, b1,
           W2, b2, iv):
    packed = _repack_tc(emb_table.T)
    packed_rows = packed.reshape(-1).reshape(NROWS, DIM // 2)
    ivf = iv.reshape(-1).astype(jnp.int32)
    iv_rep = _get_dan_sc()(ivf, packed_rows)
    small = jnp.concatenate(
        [read_depth, covariates, iv_rep, extra_features], axis=1)
    small = jnp.pad(small, ((0, 0), (0, SPAD - SMALL)))
    w1a = W1[:OOV]
    w1b = jnp.pad(W1[OOV:], ((0, SPAD - SMALL), (0, 0)))
    return _mlp_tc(oov, small, w1a, w1b, b1.reshape(1, -1), W2,
                   b2.reshape(1, -1))


# speedup vs baseline: 6.6048x; 1.1791x over previous
import functools

import jax
import jax.numpy as jnp
from jax import lax
from jax.experimental import pallas as pl
from jax.experimental.pallas import tpu as pltpu
from jax.experimental.pallas import tpu_sc as plsc

B = 4096
L = 200
DIM = 32
OOV = 512
NCOV = 8
NX = 24
HID = 512
NTOP = 64
VOCAB1 = 1000001

NC, NS = 2, 16
NW = NC * NS
RPW = B // NW
IDX0, IDX1 = 128, L - 128

TPW = 16384
NBLK = (VOCAB1 + TPW - 1) // TPW
NROWS = NBLK * TPW
CH = TPW // 8

SMALL = 1 + NCOV + DIM + NX
SPAD = 128
TILE = 512


def _repack_body(in_ref, out_ref):
    g = pl.program_id(0)
    xh = jnp.concatenate(
        [in_ref[0:16, CH * p:CH * (p + 1)] for p in range(8)], axis=0)
    xl = jnp.concatenate(
        [in_ref[16:32, CH * p:CH * (p + 1)] for p in range(8)], axis=0)
    col = lax.broadcasted_iota(jnp.int32, (128, CH), 1)
    p_in = lax.broadcasted_iota(jnp.int32, (128, CH), 0) // 16
    tok = g * TPW + p_in * CH + col
    valid = tok < VOCAB1
    xh = jnp.where(valid, xh, 0.0)
    xl = jnp.where(valid, xl, 0.0)

    row = lax.broadcasted_iota(jnp.int32, (128, 128), 0)
    lcol = lax.broadcasted_iota(jnp.int32, (128, 128), 1)
    eye = (row == lcol).astype(jnp.float32)
    a = lax.dot_general(xh, eye, (((0,), (0,)), ((), ())),
                        preferred_element_type=jnp.float32)
    bm = lax.dot_general(xl, eye, (((0,), (0,)), ((), ())),
                         preferred_element_type=jnp.float32)
    ua = lax.bitcast_convert_type(a, jnp.int32)
    ub = lax.bitcast_convert_type(bm, jnp.int32)
    hi = (ua + jnp.int32(0x8000)) & jnp.int32(-65536)
    lo = lax.shift_right_logical(ub + jnp.int32(0x8000), 16)
    out_ref[...] = lax.bitcast_convert_type(hi | lo, jnp.float32)


def _repack_tc(tblT):
    return pl.pallas_call(
        _repack_body,
        grid=(NBLK,),
        in_specs=[pl.BlockSpec((32, TPW), lambda g: (0, g))],
        out_specs=pl.BlockSpec((CH, 128), lambda g: (g, 0)),
        out_shape=jax.ShapeDtypeStruct((NBLK * CH, 128), jnp.float32),
        compiler_params=pltpu.CompilerParams(
            fuse_transposed_lhs_in_matmul=True),
    )(tblT)


def _dan_body(iv_hbm, table_hbm, out_hbm, idx_v, rows_v, out_v, sem0, sem1):
    wid = lax.axis_index("s") * NC + lax.axis_index("c")
    base = wid * (RPW * L)
    pltpu.sync_copy(iv_hbm.at[pl.ds(base, RPW * L)], idx_v.at[pl.ds(0, RPW * L)])

    ch_shift = CH.bit_length() - 1

    def xform(k, carry):
        t = idx_v[pl.ds(16 * k, 16)]
        row = ((t & jnp.int32(~(TPW - 1)))
               + ((t & jnp.int32(CH - 1)) << 3)
               + ((t >> ch_shift) & jnp.int32(7)))
        idx_v[pl.ds(16 * k, 16)] = row
        return carry

    lax.fori_loop(0, (RPW * L) // 16, xform, 0, unroll=8)

    def gather_descs(r, slot_ref, sem):
        off = r * L
        d0 = pltpu.make_async_copy(
            table_hbm.at[idx_v.at[pl.ds(off, IDX0)]],
            slot_ref.at[pl.ds(0, IDX0)], sem)
        d1 = pltpu.make_async_copy(
            table_hbm.at[idx_v.at[pl.ds(off + IDX0, IDX1)]],
            slot_ref.at[pl.ds(IDX0, IDX1)], sem)
        return d0, d1

    def issue(r, slot_ref, sem):
        d0, d1 = gather_descs(r, slot_ref, sem)
        d0.start()
        d1.start()

    lanes = lax.iota(jnp.int32, 16)

    def process(r, slot_ref, sem):
        d0, d1 = gather_descs(r, slot_ref, sem)
        d0.wait()
        d1.wait()
        ivoff = r * L
        accc = jnp.zeros((16,), jnp.int32)
        for k in range(12):
            v = idx_v[pl.ds(ivoff + 16 * k, 16)]
            accc = accc + plsc.all_reduce_population_count(v > 0)
        v = idx_v[pl.ds(ivoff + 192, 16)]
        accc = accc + plsc.all_reduce_population_count(
            (v > 0) & (lanes < L - 192))
        inv = 1.0 / jnp.maximum(accc.astype(jnp.float32), 1.0)

        def add_row(j, accs):
            a0, a1 = accs
            w = plsc.bitcast(slot_ref[j, pl.ds(0, 16)], jnp.int32)
            a0 = a0 + plsc.bitcast(w & jnp.int32(-65536), jnp.float32)
            a1 = a1 + plsc.bitcast(w << 16, jnp.float32)
            return (a0, a1)

        acc0 = jnp.zeros((16,), jnp.float32)
        acc1 = jnp.zeros((16,), jnp.float32)
        acc0, acc1 = lax.fori_loop(0, L, add_row, (acc0, acc1), unroll=8)
        out_v[r, pl.ds(0, 16)] = acc0 * inv
        out_v[r, pl.ds(16, 16)] = acc1 * inv

    issue(0, rows_v.at[0], sem0)
    issue(1, rows_v.at[1], sem1)

    def pair_body(i, carry):
        r0 = 2 * i
        process(r0, rows_v.at[0], sem0)

        @pl.when(r0 + 2 < RPW)
        def _():
            issue(r0 + 2, rows_v.at[0], sem0)

        process(r0 + 1, rows_v.at[1], sem1)

        @pl.when(r0 + 3 < RPW)
        def _():
            issue(r0 + 3, rows_v.at[1], sem1)

        return carry

    lax.fori_loop(0, RPW // 2, pair_body, 0)
    pltpu.sync_copy(out_v, out_hbm.at[pl.ds(wid * RPW, RPW)])


@functools.cache
def _get_dan_sc():
    return pl.kernel(
        _dan_body,
        out_type=jax.ShapeDtypeStruct((B, DIM), jnp.float32),
        mesh=plsc.VectorSubcoreMesh(core_axis_name="c", subcore_axis_name="s",
                                    num_cores=NC, num_subcores=NS),
        compiler_params=pltpu.CompilerParams(needs_layout_passes=False,
                                             use_tc_tiling_on_sc=False),
        scratch_types=[
            pltpu.VMEM((RPW * L + 8,), jnp.int32),
            pltpu.VMEM((2, L, DIM // 2), jnp.float32),
            pltpu.VMEM((RPW, DIM), jnp.float32),
            pltpu.SemaphoreType.DMA,
            pltpu.SemaphoreType.DMA,
        ],
    )


def _mlp_body(oov_ref, small_ref, w1a_ref, w1b_ref, b1_ref, w2_ref, b2_ref,
              out_ref):
    small = small_ref[...]
    col = lax.broadcasted_iota(jnp.int32, small.shape, 1)
    small = jnp.where(col == 0, jnp.log1p(small), small)
    x = jnp.dot(oov_ref[...], w1a_ref[...], preferred_element_type=jnp.float32)
    x = x + jnp.dot(small, w1b_ref[...], preferred_element_type=jnp.float32)
    h = jnp.maximum(x + b1_ref[...], 0.0)
    out_ref[...] = (
        lax.dot_general(w2_ref[...], h, (((0,), (1,)), ((), ())),
                        preferred_element_type=jnp.float32)
        + b2_ref[...])


def _mlp_tc(oov, small, w1a, w1b, b1, w2, b2):
    grid = (B // TILE,)
    return pl.pallas_call(
        _mlp_body,
        grid=grid,
        in_specs=[
            pl.BlockSpec((TILE, OOV), lambda i: (i, 0)),
            pl.BlockSpec((TILE, SPAD), lambda i: (i, 0)),
            pl.BlockSpec((OOV, HID), lambda i: (0, 0)),
            pl.BlockSpec((SPAD, HID), lambda i: (0, 0)),
            pl.BlockSpec((1, HID), lambda i: (0, 0)),
            pl.BlockSpec((HID, NTOP), lambda i: (0, 0)),
            pl.BlockSpec((NTOP, 1), lambda i: (0, 0)),
        ],
        out_specs=pl.BlockSpec((NTOP, TILE), lambda i: (0, i)),
        out_shape=jax.ShapeDtypeStruct((NTOP, B), jnp.float32),
    )(oov, small, w1a, w1b, b1, w2, b2)


def kernel(oov, read_depth, covariates, extra_features, emb_table, W1, b1,
           W2, b2, iv):
    packed = _repack_tc(emb_table.T)
    packed_rows = packed.reshape(-1).reshape(NROWS, DIM // 2)
    ivf = iv.reshape(-1).astype(jnp.int32)
    iv_rep = _get_dan_sc()(ivf, packed_rows)
    small = jnp.concatenate(
        [read_depth, covariates, iv_rep, extra_features], axis=1)
    small = jnp.pad(small, ((0, 0), (0, SPAD - SMALL)))
    w1a = W1[:OOV]
    w1b = jnp.pad(W1[OOV:], ((0, SPAD - SMALL), (0, 0)))
    out_t = _mlp_tc(oov, small, w1a, w1b, b1.reshape(1, -1), W2,
                    b2.reshape(-1, 1))
    return out_t.T


# speedup vs baseline: 7.7986x; 1.1808x over previous
import functools

import jax
import jax.numpy as jnp
from jax import lax
from jax.experimental import pallas as pl
from jax.experimental.pallas import tpu as pltpu
from jax.experimental.pallas import tpu_sc as plsc

B = 4096
L = 200
DIM = 32
OOV = 512
NCOV = 8
NX = 24
HID = 512
NTOP = 64
VOCAB1 = 1000001

NC, NS = 2, 16
NW = NC * NS
RPW = B // NW
IDX0, IDX1 = 128, L - 128

TPW = 16384
NBLK = (VOCAB1 + TPW - 1) // TPW
NROWS = NBLK * TPW
CH = TPW // 8

SMALL = 1 + NCOV + DIM + NX
SPAD = 128
TILE = 512


def _repack_body(in_ref, out_ref):
    g = pl.program_id(0)
    xh = jnp.concatenate(
        [in_ref[0:16, CH * p:CH * (p + 1)] for p in range(8)], axis=0)
    xl = jnp.concatenate(
        [in_ref[16:32, CH * p:CH * (p + 1)] for p in range(8)], axis=0)
    col = lax.broadcasted_iota(jnp.int32, (128, CH), 1)
    p_in = lax.broadcasted_iota(jnp.int32, (128, CH), 0) // 16
    tok = g * TPW + p_in * CH + col
    valid = tok < VOCAB1
    xh = jnp.where(valid, xh, 0.0)
    xl = jnp.where(valid, xl, 0.0)

    row = lax.broadcasted_iota(jnp.int32, (128, 128), 0)
    lcol = lax.broadcasted_iota(jnp.int32, (128, 128), 1)
    eye = (row == lcol).astype(jnp.float32)
    a = lax.dot_general(xh, eye, (((0,), (0,)), ((), ())),
                        preferred_element_type=jnp.float32)
    bm = lax.dot_general(xl, eye, (((0,), (0,)), ((), ())),
                         preferred_element_type=jnp.float32)
    ua = lax.bitcast_convert_type(a, jnp.int32)
    ub = lax.bitcast_convert_type(bm, jnp.int32)
    hi = (ua + jnp.int32(0x8000)) & jnp.int32(-65536)
    lo = lax.shift_right_logical(ub + jnp.int32(0x8000), 16)
    out_ref[...] = lax.bitcast_convert_type(hi | lo, jnp.float32)


def _repack_tc(tblT):
    return pl.pallas_call(
        _repack_body,
        grid=(NBLK,),
        in_specs=[pl.BlockSpec((32, TPW), lambda g: (0, g))],
        out_specs=pl.BlockSpec((CH, 128), lambda g: (g, 0)),
        out_shape=jax.ShapeDtypeStruct((NBLK * CH, 128), jnp.float32),
        compiler_params=pltpu.CompilerParams(
            fuse_transposed_lhs_in_matmul=True),
    )(tblT)


def _dan_body(iv_hbm, table_hbm, out_hbm, idx_v, rows_v, out_v,
              sem0, sem1, sem2, sem3):
    wid = lax.axis_index("s") * NC + lax.axis_index("c")
    base = wid * (RPW * L)
    pltpu.sync_copy(iv_hbm.at[pl.ds(base, RPW * L)], idx_v.at[pl.ds(0, RPW * L)])

    ch_shift = CH.bit_length() - 1

    def xform(k, carry):
        t = idx_v[pl.ds(16 * k, 16)]
        row = ((t & jnp.int32(~(TPW - 1)))
               + ((t & jnp.int32(CH - 1)) << 3)
               + ((t >> ch_shift) & jnp.int32(7)))
        idx_v[pl.ds(16 * k, 16)] = row
        return carry

    lax.fori_loop(0, (RPW * L) // 16, xform, 0, unroll=8)

    def gather_descs(r, slot_ref, sem):
        off = r * L
        d0 = pltpu.make_async_copy(
            table_hbm.at[idx_v.at[pl.ds(off, IDX0)]],
            slot_ref.at[pl.ds(0, IDX0)], sem)
        d1 = pltpu.make_async_copy(
            table_hbm.at[idx_v.at[pl.ds(off + IDX0, IDX1)]],
            slot_ref.at[pl.ds(IDX0, IDX1)], sem)
        return d0, d1

    def issue(r, slot_ref, sem):
        d0, d1 = gather_descs(r, slot_ref, sem)
        d0.start()
        d1.start()

    lanes = lax.iota(jnp.int32, 16)

    def process(r, slot_ref, sem):
        d0, d1 = gather_descs(r, slot_ref, sem)
        d0.wait()
        d1.wait()
        ivoff = r * L
        accc = jnp.zeros((16,), jnp.int32)
        for k in range(12):
            v = idx_v[pl.ds(ivoff + 16 * k, 16)]
            accc = accc + plsc.all_reduce_population_count(v > 0)
        v = idx_v[pl.ds(ivoff + 192, 16)]
        accc = accc + plsc.all_reduce_population_count(
            (v > 0) & (lanes < L - 192))
        inv = 1.0 / jnp.maximum(accc.astype(jnp.float32), 1.0)

        def add_row(j, accs):
            a0, a1 = accs
            w = plsc.bitcast(slot_ref[j, pl.ds(0, 16)], jnp.int32)
            a0 = a0 + plsc.bitcast(w & jnp.int32(-65536), jnp.float32)
            a1 = a1 + plsc.bitcast(w << 16, jnp.float32)
            return (a0, a1)

        acc0 = jnp.zeros((16,), jnp.float32)
        acc1 = jnp.zeros((16,), jnp.float32)
        acc0, acc1 = lax.fori_loop(0, L, add_row, (acc0, acc1), unroll=8)
        out_v[r, pl.ds(0, 16)] = acc0 * inv
        out_v[r, pl.ds(16, 16)] = acc1 * inv

    sems = (sem0, sem1, sem2, sem3)
    for j in range(4):
        issue(j, rows_v.at[j], sems[j])

    def quad_body(i, carry):
        r0 = 4 * i
        for j in range(4):
            process(r0 + j, rows_v.at[j], sems[j])

            @pl.when(r0 + j + 4 < RPW)
            def _():
                issue(r0 + j + 4, rows_v.at[j], sems[j])

        return carry

    lax.fori_loop(0, RPW // 4, quad_body, 0)
    pltpu.sync_copy(out_v, out_hbm.at[pl.ds(wid * RPW, RPW)])


@functools.cache
def _get_dan_sc():
    return pl.kernel(
        _dan_body,
        out_type=jax.ShapeDtypeStruct((B, DIM), jnp.float32),
        mesh=plsc.VectorSubcoreMesh(core_axis_name="c", subcore_axis_name="s",
                                    num_cores=NC, num_subcores=NS),
        compiler_params=pltpu.CompilerParams(needs_layout_passes=False,
                                             use_tc_tiling_on_sc=False),
        scratch_types=[
            pltpu.VMEM((RPW * L + 8,), jnp.int32),
            pltpu.VMEM((4, L, DIM // 2), jnp.float32),
            pltpu.VMEM((RPW, DIM), jnp.float32),
            pltpu.SemaphoreType.DMA,
            pltpu.SemaphoreType.DMA,
            pltpu.SemaphoreType.DMA,
            pltpu.SemaphoreType.DMA,
        ],
    )


def _mlp_body(oov_ref, small_ref, w1a_ref, w1b_ref, b1_ref, w2_ref, b2_ref,
              out_ref):
    small = small_ref[...]
    col = lax.broadcasted_iota(jnp.int32, small.shape, 1)
    small = jnp.where(col == 0, jnp.log1p(small), small)
    x = jnp.dot(oov_ref[...], w1a_ref[...], preferred_element_type=jnp.float32)
    x = x + jnp.dot(small, w1b_ref[...], preferred_element_type=jnp.float32)
    h = jnp.maximum(x + b1_ref[...], 0.0)
    out_ref[...] = (
        lax.dot_general(w2_ref[...], h, (((0,), (1,)), ((), ())),
                        preferred_element_type=jnp.float32)
        + b2_ref[...])


def _mlp_tc(oov, small, w1a, w1b, b1, w2, b2):
    grid = (B // TILE,)
    return pl.pallas_call(
        _mlp_body,
        grid=grid,
        in_specs=[
            pl.BlockSpec((TILE, OOV), lambda i: (i, 0)),
            pl.BlockSpec((TILE, SPAD), lambda i: (i, 0)),
            pl.BlockSpec((OOV, HID), lambda i: (0, 0)),
            pl.BlockSpec((SPAD, HID), lambda i: (0, 0)),
            pl.BlockSpec((1, HID), lambda i: (0, 0)),
            pl.BlockSpec((HID, NTOP), lambda i: (0, 0)),
            pl.BlockSpec((NTOP, 1), lambda i: (0, 0)),
        ],
        out_specs=pl.BlockSpec((NTOP, TILE), lambda i: (0, i)),
        out_shape=jax.ShapeDtypeStruct((NTOP, B), jnp.float32),
    )(oov, small, w1a, w1b, b1, w2, b2)


def kernel(oov, read_depth, covariates, extra_features, emb_table, W1, b1,
           W2, b2, iv):
    packed = _repack_tc(emb_table.T)
    packed_rows = packed.reshape(-1).reshape(NROWS, DIM // 2)
    ivf = iv.reshape(-1).astype(jnp.int32)
    iv_rep = _get_dan_sc()(ivf, packed_rows)
    small = jnp.concatenate(
        [read_depth, covariates, iv_rep, extra_features], axis=1)
    small = jnp.pad(small, ((0, 0), (0, SPAD - SMALL)))
    w1a = W1[:OOV]
    w1b = jnp.pad(W1[OOV:], ((0, SPAD - SMALL), (0, 0)))
    out_t = _mlp_tc(oov, small, w1a, w1b, b1.reshape(1, -1), W2,
                    b2.reshape(-1, 1))
    return out_t.T
